# baseline (speedup 1.0000x reference)
import functools

import jax
import jax.numpy as jnp
from jax import lax
from jax.experimental import pallas as pl
from jax.experimental.pallas import tpu as pltpu
from jax.experimental.pallas import tpu_sc as plsc

N = 10000
E = 320000
D = 128
H = 128

NC = 2
NS = 16
NW = NC * NS
K = 128
NCHUNK = E // K
BPW = 80
NCH_PAD = BPW * NW
ZRP = 1000
ZNS = N // ZRP

_mesh = plsc.VectorSubcoreMesh(core_axis_name="c", subcore_axis_name="s")


NP = 10240
DRP = NP // NS


@functools.partial(
    pl.kernel,
    out_type=jax.ShapeDtypeStruct((NC, 1, NP), jnp.float32),
    mesh=_mesh,
    scratch_types=[
        pltpu.VMEM((1, K), jnp.int32),
        pltpu.VMEM((1, K), jnp.int32),
        pltpu.VMEM((K,), jnp.float32),
        pltpu.VMEM_SHARED((NP,), jnp.float32),
        pltpu.SemaphoreType.DMA,
        pltpu.SemaphoreType.DMA,
    ],
)
def _sc_degree(eflat_hbm, ones_hbm, zeros_hbm, out_hbm,
               didx_a, didx_b, ones_v, acc, sem_da, sem_db):
    c = lax.axis_index("c")
    s = lax.axis_index("s")
    wid = c * NS + s
    cnt = jnp.where(wid < NW - 1, BPW, NCHUNK - BPW * (NW - 1))

    pltpu.sync_copy(ones_hbm, ones_v)

    pltpu.sync_copy(zeros_hbm.at[0, pl.ds(s * DRP, DRP)],
                    acc.at[pl.ds(s * DRP, DRP)])
    plsc.subcore_barrier()

    def dfetch(j, didx, sem):
        pltpu.async_copy(
            eflat_hbm.at[pl.ds(E + (wid * BPW + j) * K, K)], didx.at[0], sem
        )

    def dwait(didx, sem):
        pltpu.make_async_copy(
            eflat_hbm.at[pl.ds(0, K)], didx.at[0], sem
        ).wait()

    def scat(didx):
        pltpu.sync_copy(ones_v, acc.at[didx.at[0]], add=True)

    dfetch(0, didx_a, sem_da)

    @pl.loop(0, cnt // 2)
    def _(p):
        j = 2 * p
        dfetch(j + 1, didx_b, sem_db)
        dwait(didx_a, sem_da)
        scat(didx_a)

        @pl.when(j + 2 < cnt)
        def _():
            dfetch(j + 2, didx_a, sem_da)

        dwait(didx_b, sem_db)
        scat(didx_b)

    plsc.subcore_barrier()
    pltpu.sync_copy(acc.at[pl.ds(s * DRP, DRP)],
                    out_hbm.at[c].at[0, pl.ds(s * DRP, DRP)])


@functools.partial(
    pl.kernel,
    out_type=jax.ShapeDtypeStruct((NC, N, H), jnp.float32),
    mesh=_mesh,
    scratch_types=[
        pltpu.VMEM((BPW * K,), jnp.int32),
        pltpu.VMEM((1, K), jnp.int32),
        pltpu.VMEM((1, K), jnp.int32),
        pltpu.VMEM((K, H), jnp.float32),
        pltpu.VMEM((K, H), jnp.float32),
        pltpu.VMEM_SHARED((N, H), jnp.float32),
        pltpu.SemaphoreType.DMA,
        pltpu.SemaphoreType.DMA,
        pltpu.SemaphoreType.DMA,
        pltpu.SemaphoreType.DMA,
    ],
)
def _sc_edge_agg(xn_hbm, eflat_hbm, zeros_hbm, out_hbm,
                 sidx_all, didx_a, didx_b, rows_a, rows_b, acc,
                 sem_a, sem_b, sem_da, sem_db):
    c = lax.axis_index("c")
    s = lax.axis_index("s")
    wid = c * NS + s
    cnt = jnp.where(wid < NW - 1, BPW, NCHUNK - BPW * (NW - 1))

    pltpu.sync_copy(eflat_hbm.at[pl.ds(wid * BPW * K, BPW * K)], sidx_all)

    @pl.when(s < ZNS)
    def _():
        pltpu.sync_copy(
            zeros_hbm.at[pl.ds(s * ZRP, ZRP)], acc.at[pl.ds(s * ZRP, ZRP)]
        )

    plsc.subcore_barrier()

    def gather(j, rows, sem):
        pltpu.async_copy(xn_hbm.at[sidx_all.at[pl.ds(j * K, K)]], rows, sem)

    def gwait(rows, sem):
        pltpu.make_async_copy(
            xn_hbm.at[sidx_all.at[pl.ds(0, K)]], rows, sem
        ).wait()

    def dfetch(j, didx, sem):
        pltpu.async_copy(
            eflat_hbm.at[pl.ds(E + (wid * BPW + j) * K, K)], didx.at[0], sem
        )

    def dwait(j, didx, sem):
        pltpu.make_async_copy(
            eflat_hbm.at[pl.ds(0, K)], didx.at[0], sem
        ).wait()

    def scat(rows, didx):
        pltpu.sync_copy(rows, acc.at[didx.at[0]], add=True)

    dfetch(0, didx_a, sem_da)
    gather(0, rows_a, sem_a)

    @pl.loop(0, cnt // 2)
    def _(p):
        j = 2 * p
        dfetch(j + 1, didx_b, sem_db)
        gather(j + 1, rows_b, sem_b)
        gwait(rows_a, sem_a)
        dwait(j, didx_a, sem_da)
        scat(rows_a, didx_a)

        @pl.when(j + 2 < cnt)
        def _():
            dfetch(j + 2, didx_a, sem_da)
            gather(j + 2, rows_a, sem_a)

        gwait(rows_b, sem_b)
        dwait(j + 1, didx_b, sem_db)
        scat(rows_b, didx_b)

    plsc.subcore_barrier()

    @pl.when(s < ZNS)
    def _():
        pltpu.sync_copy(
            acc.at[pl.ds(s * ZRP, ZRP)], out_hbm.at[c].at[pl.ds(s * ZRP, ZRP)]
        )


def _tc_proj_body(f_ref, t_ref, wp_ref, bp_ref, wd_ref, bd_ref, h_ref):
    x = f_ref[...]
    proj_p = jnp.dot(x, wp_ref[...], preferred_element_type=jnp.float32) + bp_ref[...]
    proj_d = jnp.dot(x, wd_ref[...], preferred_element_type=jnp.float32) + bd_ref[...]
    h_ref[...] = jnp.where(t_ref[...] == 0, proj_p, proj_d)


def _tc_scale_body(h_ref, degp_ref, xn_ref, norm_ref):
    deg = degp_ref[:, 0:1] + degp_ref[:, 1:2]
    norm = lax.rsqrt(jnp.maximum(deg, 1.0))
    xn_ref[...] = h_ref[...] * norm
    norm_ref[...] = norm


def _tc_layer_body(aggp_ref, norm_ref, w_ref, b_ref, out_ref, *, relu, rescale):
    norm = norm_ref[...]
    agg = (aggp_ref[0] + aggp_ref[1]) * norm
    y = jnp.dot(agg, w_ref[...], preferred_element_type=jnp.float32) + b_ref[...]
    if relu:
        y = jnp.maximum(y, 0.0)
    if rescale:
        y = y * norm
    out_ref[...] = y


def kernel(features, edge_index, node_types, W_person, b_person,
           W_disease, b_disease, W1, b1, W2, b2):
    zeros_h = jnp.zeros((N, H), jnp.float32)
    zeros_np = jnp.zeros((1, NP), jnp.float32)
    types2d = node_types.reshape(N, 1)
    bp = b_person.reshape(1, H)
    bd = b_disease.reshape(1, H)
    b1r = b1.reshape(1, H)
    b2r = b2.reshape(1, H)

    eflat = edge_index.reshape(2 * E)

    degp = _sc_degree(eflat, jnp.ones((K,), jnp.float32), zeros_np)
    degt = degp.reshape(NC, NP)[:, :N].T

    h0 = pl.pallas_call(
        _tc_proj_body,
        out_shape=jax.ShapeDtypeStruct((N, H), jnp.float32),
    )(features, types2d, W_person, bp, W_disease, bd)

    xn1, norm = pl.pallas_call(
        _tc_scale_body,
        out_shape=(
            jax.ShapeDtypeStruct((N, H), jnp.float32),
            jax.ShapeDtypeStruct((N, 1), jnp.float32),
        ),
    )(h0, degt)

    aggp1 = _sc_edge_agg(xn1, eflat, zeros_h)

    xn2 = pl.pallas_call(
        functools.partial(_tc_layer_body, relu=True, rescale=True),
        out_shape=jax.ShapeDtypeStruct((N, H), jnp.float32),
    )(aggp1, norm, W1, b1r)

    aggp2 = _sc_edge_agg(xn2, eflat, zeros_h)

    out = pl.pallas_call(
        functools.partial(_tc_layer_body, relu=False, rescale=False),
        out_shape=jax.ShapeDtypeStruct((N, H), jnp.float32),
    )(aggp2, norm, W2, b2r)

    return out


# speedup vs baseline: 1.0136x; 1.0136x over previous
import functools

import jax
import jax.numpy as jnp
from jax import lax
from jax.experimental import pallas as pl
from jax.experimental.pallas import tpu as pltpu
from jax.experimental.pallas import tpu_sc as plsc

N = 10000
E = 320000
D = 128
H = 128

NC = 2
NS = 16
NW = NC * NS
K = 128
NCHUNK = E // K
BPW = 80
NCH_PAD = BPW * NW
ZRP = 1000
ZNS = N // ZRP

_mesh = plsc.VectorSubcoreMesh(core_axis_name="c", subcore_axis_name="s")


NP = 10240
DRP = NP // NS


@functools.partial(
    pl.kernel,
    out_type=jax.ShapeDtypeStruct((NC, 1, NP), jnp.float32),
    mesh=_mesh,
    scratch_types=[
        pltpu.VMEM((2, K), jnp.int32),
        pltpu.VMEM((2, K), jnp.int32),
        pltpu.VMEM((K,), jnp.float32),
        pltpu.VMEM_SHARED((NP,), jnp.float32),
        pltpu.SemaphoreType.DMA,
        pltpu.SemaphoreType.DMA,
    ],
)
def _sc_degree(edge_hbm, ones_hbm, zeros_hbm, out_hbm,
               didx_a, didx_b, ones_v, acc, sem_da, sem_db):
    c = lax.axis_index("c")
    s = lax.axis_index("s")
    wid = c * NS + s
    cnt = jnp.where(wid < NW - 1, BPW, NCHUNK - BPW * (NW - 1))

    pltpu.sync_copy(ones_hbm, ones_v)

    pltpu.sync_copy(zeros_hbm.at[0, pl.ds(s * DRP, DRP)],
                    acc.at[pl.ds(s * DRP, DRP)])
    plsc.subcore_barrier()

    def dfetch(j, didx, sem):
        pltpu.async_copy(
            edge_hbm.at[:, pl.ds((wid * BPW + j) * K, K)], didx, sem
        )

    def dwait(didx, sem):
        pltpu.make_async_copy(
            edge_hbm.at[:, pl.ds(0, K)], didx, sem
        ).wait()

    def scat(didx):
        pltpu.sync_copy(ones_v, acc.at[didx.at[1]], add=True)

    dfetch(0, didx_a, sem_da)

    @pl.loop(0, cnt // 2)
    def _(p):
        j = 2 * p
        dfetch(j + 1, didx_b, sem_db)
        dwait(didx_a, sem_da)
        scat(didx_a)

        @pl.when(j + 2 < cnt)
        def _():
            dfetch(j + 2, didx_a, sem_da)

        dwait(didx_b, sem_db)
        scat(didx_b)

    plsc.subcore_barrier()
    pltpu.sync_copy(acc.at[pl.ds(s * DRP, DRP)],
                    out_hbm.at[c].at[0, pl.ds(s * DRP, DRP)])


@functools.partial(
    pl.kernel,
    out_type=jax.ShapeDtypeStruct((NC, N, H), jnp.float32),
    mesh=_mesh,
    scratch_types=[
        [pltpu.VMEM((2, K), jnp.int32)] * 4,
        pltpu.VMEM((K, H), jnp.float32),
        pltpu.VMEM((K, H), jnp.float32),
        pltpu.VMEM_SHARED((N, H), jnp.float32),
        [pltpu.SemaphoreType.DMA] * 4,
        pltpu.SemaphoreType.DMA,
        pltpu.SemaphoreType.DMA,
    ],
)
def _sc_edge_agg(xn_hbm, edge_hbm, zeros_hbm, out_hbm,
                 eidx, rows_a, rows_b, acc, esem, sem_a, sem_b):
    c = lax.axis_index("c")
    s = lax.axis_index("s")
    wid = c * NS + s
    cnt = jnp.where(wid < NW - 1, BPW, NCHUNK - BPW * (NW - 1))

    @pl.when(s < ZNS)
    def _():
        pltpu.sync_copy(
            zeros_hbm.at[pl.ds(s * ZRP, ZRP)], acc.at[pl.ds(s * ZRP, ZRP)]
        )

    plsc.subcore_barrier()

    def efetch(j, q):
        pltpu.async_copy(
            edge_hbm.at[:, pl.ds((wid * BPW + j) * K, K)], eidx[q], esem[q]
        )

    def ewait(q):
        pltpu.make_async_copy(
            edge_hbm.at[:, pl.ds(0, K)], eidx[q], esem[q]
        ).wait()

    def gather(q, rows, sem):
        pltpu.async_copy(xn_hbm.at[eidx[q].at[0]], rows, sem)

    def gwait(q, rows, sem):
        pltpu.make_async_copy(xn_hbm.at[eidx[q].at[0]], rows, sem).wait()

    def scat(rows, q):
        pltpu.sync_copy(rows, acc.at[eidx[q].at[1]], add=True)

    def refill(j, q):
        @pl.when(j < cnt)
        def _():
            efetch(j, q)

    for q in range(4):
        efetch(q, q)
    ewait(0)
    gather(0, rows_a, sem_a)
    ewait(1)
    gather(1, rows_b, sem_b)

    @pl.loop(0, cnt // 4)
    def _(it):
        j = 4 * it
        gwait(0, rows_a, sem_a)
        scat(rows_a, 0)
        refill(j + 4, 0)
        ewait(2)
        gather(2, rows_a, sem_a)
        gwait(1, rows_b, sem_b)
        scat(rows_b, 1)
        refill(j + 5, 1)
        ewait(3)
        gather(3, rows_b, sem_b)
        gwait(2, rows_a, sem_a)
        scat(rows_a, 2)
        refill(j + 6, 2)

        @pl.when(j + 4 < cnt)
        def _():
            ewait(0)
            gather(0, rows_a, sem_a)

        gwait(3, rows_b, sem_b)
        scat(rows_b, 3)
        refill(j + 7, 3)

        @pl.when(j + 5 < cnt)
        def _():
            ewait(1)
            gather(1, rows_b, sem_b)

    plsc.subcore_barrier()

    @pl.when(s < ZNS)
    def _():
        pltpu.sync_copy(
            acc.at[pl.ds(s * ZRP, ZRP)], out_hbm.at[c].at[pl.ds(s * ZRP, ZRP)]
        )


def _tc_proj_body(f_ref, t_ref, wp_ref, bp_ref, wd_ref, bd_ref, h_ref):
    x = f_ref[...]
    proj_p = jnp.dot(x, wp_ref[...], preferred_element_type=jnp.float32) + bp_ref[...]
    proj_d = jnp.dot(x, wd_ref[...], preferred_element_type=jnp.float32) + bd_ref[...]
    h_ref[...] = jnp.where(t_ref[...] == 0, proj_p, proj_d)


def _tc_scale_body(h_ref, degp_ref, xn_ref, norm_ref):
    deg = degp_ref[:, 0:1] + degp_ref[:, 1:2]
    norm = lax.rsqrt(jnp.maximum(deg, 1.0))
    xn_ref[...] = h_ref[...] * norm
    norm_ref[...] = norm


def _tc_layer_body(aggp_ref, norm_ref, w_ref, b_ref, out_ref, *, relu, rescale):
    norm = norm_ref[...]
    agg = (aggp_ref[0] + aggp_ref[1]) * norm
    y = jnp.dot(agg, w_ref[...], preferred_element_type=jnp.float32) + b_ref[...]
    if relu:
        y = jnp.maximum(y, 0.0)
    if rescale:
        y = y * norm
    out_ref[...] = y


def kernel(features, edge_index, node_types, W_person, b_person,
           W_disease, b_disease, W1, b1, W2, b2):
    zeros_h = jnp.zeros((N, H), jnp.float32)
    zeros_np = jnp.zeros((1, NP), jnp.float32)
    types2d = node_types.reshape(N, 1)
    bp = b_person.reshape(1, H)
    bd = b_disease.reshape(1, H)
    b1r = b1.reshape(1, H)
    b2r = b2.reshape(1, H)

    degp = _sc_degree(edge_index, jnp.ones((K,), jnp.float32), zeros_np)
    degt = degp.reshape(NC, NP)[:, :N].T

    h0 = pl.pallas_call(
        _tc_proj_body,
        out_shape=jax.ShapeDtypeStruct((N, H), jnp.float32),
    )(features, types2d, W_person, bp, W_disease, bd)

    xn1, norm = pl.pallas_call(
        _tc_scale_body,
        out_shape=(
            jax.ShapeDtypeStruct((N, H), jnp.float32),
            jax.ShapeDtypeStruct((N, 1), jnp.float32),
        ),
    )(h0, degt)

    aggp1 = _sc_edge_agg(xn1, edge_index, zeros_h)

    xn2 = pl.pallas_call(
        functools.partial(_tc_layer_body, relu=True, rescale=True),
        out_shape=jax.ShapeDtypeStruct((N, H), jnp.float32),
    )(aggp1, norm, W1, b1r)

    aggp2 = _sc_edge_agg(xn2, edge_index, zeros_h)

    out = pl.pallas_call(
        functools.partial(_tc_layer_body, relu=False, rescale=False),
        out_shape=jax.ShapeDtypeStruct((N, H), jnp.float32),
    )(aggp2, norm, W2, b2r)

    return out


# speedup vs baseline: 1.0704x; 1.0561x over previous
import functools

import jax
import jax.numpy as jnp
from jax import lax
from jax.experimental import pallas as pl
from jax.experimental.pallas import tpu as pltpu
from jax.experimental.pallas import tpu_sc as plsc

N = 10000
E = 320000
D = 128
H = 128

NC = 2
NS = 16
NW = NC * NS
K = 128
NCHUNK = E // K
BPW = 80
NBIG = 17


def _worker_range(wid):
    start = 76 * wid + 4 * jnp.minimum(wid, NBIG)
    cnt = jnp.where(wid < NBIG, 80, 76)
    return start, cnt
ZRP = 1000
ZNS = N // ZRP

_mesh = plsc.VectorSubcoreMesh(core_axis_name="c", subcore_axis_name="s")


NP = 10240
DRP = NP // NS


@functools.partial(
    pl.kernel,
    out_type=jax.ShapeDtypeStruct((NC, 1, NP), jnp.float32),
    mesh=_mesh,
    scratch_types=[
        pltpu.VMEM((2, BPW * K), jnp.int32),
        pltpu.VMEM((K,), jnp.float32),
        pltpu.VMEM_SHARED((NP,), jnp.float32),
    ],
)
def _sc_degree(edge_hbm, ones_hbm, zeros_hbm, out_hbm, ebuf, ones_v, acc):
    c = lax.axis_index("c")
    s = lax.axis_index("s")
    wid = c * NS + s
    start, cnt = _worker_range(wid)

    pltpu.sync_copy(ones_hbm, ones_v)
    bstart = jnp.minimum(start, NCHUNK - BPW)
    off = start - bstart
    pltpu.sync_copy(edge_hbm.at[:, pl.ds(bstart * K, BPW * K)], ebuf)

    pltpu.sync_copy(zeros_hbm.at[0, pl.ds(s * DRP, DRP)],
                    acc.at[pl.ds(s * DRP, DRP)])
    plsc.subcore_barrier()

    @pl.loop(0, cnt)
    def _(j):
        pltpu.sync_copy(
            ones_v, acc.at[ebuf.at[1, pl.ds((j + off) * K, K)]], add=True
        )

    plsc.subcore_barrier()
    pltpu.sync_copy(acc.at[pl.ds(s * DRP, DRP)],
                    out_hbm.at[c].at[0, pl.ds(s * DRP, DRP)])


@functools.partial(
    pl.kernel,
    out_type=jax.ShapeDtypeStruct((NC, N, H), jnp.float32),
    mesh=_mesh,
    scratch_types=[
        [pltpu.VMEM((2, K), jnp.int32)] * 4,
        pltpu.VMEM((K, H), jnp.float32),
        pltpu.VMEM((K, H), jnp.float32),
        pltpu.VMEM_SHARED((N, H), jnp.float32),
        [pltpu.SemaphoreType.DMA] * 4,
        pltpu.SemaphoreType.DMA,
        pltpu.SemaphoreType.DMA,
    ],
)
def _sc_edge_agg(xn_hbm, edge_hbm, zeros_hbm, out_hbm,
                 eidx, rows_a, rows_b, acc, esem, sem_a, sem_b):
    c = lax.axis_index("c")
    s = lax.axis_index("s")
    wid = c * NS + s
    start, cnt = _worker_range(wid)

    @pl.when(s < ZNS)
    def _():
        pltpu.sync_copy(
            zeros_hbm.at[pl.ds(s * ZRP, ZRP)], acc.at[pl.ds(s * ZRP, ZRP)]
        )

    plsc.subcore_barrier()

    def efetch(j, q):
        pltpu.async_copy(
            edge_hbm.at[:, pl.ds((start + j) * K, K)], eidx[q], esem[q]
        )

    def ewait(q):
        pltpu.make_async_copy(
            edge_hbm.at[:, pl.ds(0, K)], eidx[q], esem[q]
        ).wait()

    def gather(q, rows, sem):
        pltpu.async_copy(xn_hbm.at[eidx[q].at[0]], rows, sem)

    def gwait(q, rows, sem):
        pltpu.make_async_copy(xn_hbm.at[eidx[q].at[0]], rows, sem).wait()

    def scat(rows, q):
        pltpu.sync_copy(rows, acc.at[eidx[q].at[1]], add=True)

    def refill(j, q):
        @pl.when(j < cnt)
        def _():
            efetch(j, q)

    for q in range(4):
        efetch(q, q)
    ewait(0)
    gather(0, rows_a, sem_a)
    ewait(1)
    gather(1, rows_b, sem_b)

    @pl.loop(0, cnt // 4)
    def _(it):
        j = 4 * it
        gwait(0, rows_a, sem_a)
        scat(rows_a, 0)
        refill(j + 4, 0)
        ewait(2)
        gather(2, rows_a, sem_a)
        gwait(1, rows_b, sem_b)
        scat(rows_b, 1)
        refill(j + 5, 1)
        ewait(3)
        gather(3, rows_b, sem_b)
        gwait(2, rows_a, sem_a)
        scat(rows_a, 2)
        refill(j + 6, 2)

        @pl.when(j + 4 < cnt)
        def _():
            ewait(0)
            gather(0, rows_a, sem_a)

        gwait(3, rows_b, sem_b)
        scat(rows_b, 3)
        refill(j + 7, 3)

        @pl.when(j + 5 < cnt)
        def _():
            ewait(1)
            gather(1, rows_b, sem_b)

    plsc.subcore_barrier()

    @pl.when(s < ZNS)
    def _():
        pltpu.sync_copy(
            acc.at[pl.ds(s * ZRP, ZRP)], out_hbm.at[c].at[pl.ds(s * ZRP, ZRP)]
        )


def _tc_proj_body(f_ref, t_ref, wp_ref, bp_ref, wd_ref, bd_ref, h_ref):
    x = f_ref[...]
    proj_p = jnp.dot(x, wp_ref[...], preferred_element_type=jnp.float32) + bp_ref[...]
    proj_d = jnp.dot(x, wd_ref[...], preferred_element_type=jnp.float32) + bd_ref[...]
    h_ref[...] = jnp.where(t_ref[...] == 0, proj_p, proj_d)


def _tc_scale_body(h_ref, degp_ref, xn_ref, norm_ref):
    deg = degp_ref[:, 0:1] + degp_ref[:, 1:2]
    norm = lax.rsqrt(jnp.maximum(deg, 1.0))
    xn_ref[...] = h_ref[...] * norm
    norm_ref[...] = norm


def _tc_layer_body(aggp_ref, norm_ref, w_ref, b_ref, out_ref, *, relu, rescale):
    norm = norm_ref[...]
    agg = (aggp_ref[0] + aggp_ref[1]) * norm
    y = jnp.dot(agg, w_ref[...], preferred_element_type=jnp.float32) + b_ref[...]
    if relu:
        y = jnp.maximum(y, 0.0)
    if rescale:
        y = y * norm
    out_ref[...] = y


def kernel(features, edge_index, node_types, W_person, b_person,
           W_disease, b_disease, W1, b1, W2, b2):
    zeros_h = jnp.zeros((N, H), jnp.float32)
    zeros_np = jnp.zeros((1, NP), jnp.float32)
    types2d = node_types.reshape(N, 1)
    bp = b_person.reshape(1, H)
    bd = b_disease.reshape(1, H)
    b1r = b1.reshape(1, H)
    b2r = b2.reshape(1, H)

    degp = _sc_degree(edge_index, jnp.ones((K,), jnp.float32), zeros_np)
    degt = degp.reshape(NC, NP)[:, :N].T

    h0 = pl.pallas_call(
        _tc_proj_body,
        out_shape=jax.ShapeDtypeStruct((N, H), jnp.float32),
    )(features, types2d, W_person, bp, W_disease, bd)

    xn1, norm = pl.pallas_call(
        _tc_scale_body,
        out_shape=(
            jax.ShapeDtypeStruct((N, H), jnp.float32),
            jax.ShapeDtypeStruct((N, 1), jnp.float32),
        ),
    )(h0, degt)

    aggp1 = _sc_edge_agg(xn1, edge_index, zeros_h)

    xn2 = pl.pallas_call(
        functools.partial(_tc_layer_body, relu=True, rescale=True),
        out_shape=jax.ShapeDtypeStruct((N, H), jnp.float32),
    )(aggp1, norm, W1, b1r)

    aggp2 = _sc_edge_agg(xn2, edge_index, zeros_h)

    out = pl.pallas_call(
        functools.partial(_tc_layer_body, relu=False, rescale=False),
        out_shape=jax.ShapeDtypeStruct((N, H), jnp.float32),
    )(aggp2, norm, W2, b2r)

    return out
